# Initial kernel scaffold; baseline (speedup 1.0000x reference)
import jax
import jax.numpy as jnp
from jax.experimental import pallas as pl


def kernel(x, edge_index, batch, W1, a_src1, a_dst1, b1, W2, a_src2, a_dst2, b2, Wl1, bl1, Wl2, bl2):
    raise NotImplementedError("write your pallas kernel here")


# speedup vs baseline: 12.5682x; 12.5682x over previous
import jax
import jax.numpy as jnp
from jax import lax
from jax.experimental import pallas as pl
from jax.experimental.pallas import tpu as pltpu
from jax.experimental.pallas import tpu_sc as plsc

N = 10000
E = 320000
D = 128
DH = D // 2
HID = 64
DOUT = 32
NG = 64

NC, NS = 2, 16
EB = 512
NBLK = 42
EW = EB * NBLK
ET = EW * NS
ER = ET // 128
NACC = 10240
STRIPE = NACC // NS

_f32 = jnp.float32


def _selu(x):
    return 1.0507009873554805 * jnp.where(
        x > 0, x, 1.6732632423543772 * (jnp.exp(x) - 1.0))


def _mm_body(x_ref, w_ref, a2_ref, h_ref, p_ref):
    h = jnp.dot(x_ref[...], w_ref[...], preferred_element_type=_f32)
    h_ref[0] = h[:, :DH]
    h_ref[1] = h[:, DH:]
    p_ref[...] = jnp.dot(h, a2_ref[...], preferred_element_type=_f32)


def _mm(x, W, A2):
    BR = 1024
    return pl.pallas_call(
        _mm_body,
        grid=(NACC // BR,),
        in_specs=[
            pl.BlockSpec((BR, D), lambda i: (i, 0)),
            pl.BlockSpec((D, D), lambda i: (0, 0)),
            pl.BlockSpec((D, 2), lambda i: (0, 0)),
        ],
        out_specs=[
            pl.BlockSpec((NC, BR, DH), lambda i: (0, i, 0)),
            pl.BlockSpec((BR, 2), lambda i: (i, 0)),
        ],
        out_shape=[
            jax.ShapeDtypeStruct((NC, NACC, DH), _f32),
            jax.ShapeDtypeStruct((NACC, 2), _f32),
        ],
    )(x, W, A2)


def _mid_body(a0_ref, a1_ref, d_ref, b_ref, w_ref, a2_ref, h_ref, p_ref):
    i = pl.program_id(0)
    den = d_ref[...] + 1e-16
    acc = jnp.concatenate([a0_ref[...], a1_ref[...]], axis=1)
    z = _selu(acc / den + b_ref[...])
    row = i * a0_ref.shape[0] + lax.broadcasted_iota(jnp.int32, z.shape, 0)
    z = jnp.where(row < N, z, 0.0)
    h = jnp.dot(z, w_ref[...], preferred_element_type=_f32)
    h_ref[0] = h[:, :DH]
    h_ref[1] = h[:, DH:]
    p_ref[...] = jnp.dot(h, a2_ref[...], preferred_element_type=_f32)


def _mid(acc, den, b, W, A2):
    BR = 1024
    return pl.pallas_call(
        _mid_body,
        grid=(NACC // BR,),
        in_specs=[
            pl.BlockSpec((BR, DH), lambda i: (i, 0)),
            pl.BlockSpec((BR, DH), lambda i: (i, 0)),
            pl.BlockSpec((BR, 1), lambda i: (i, 0)),
            pl.BlockSpec((1, D), lambda i: (0, 0)),
            pl.BlockSpec((D, D), lambda i: (0, 0)),
            pl.BlockSpec((D, 2), lambda i: (0, 0)),
        ],
        out_specs=[
            pl.BlockSpec((NC, BR, DH), lambda i: (0, i, 0)),
            pl.BlockSpec((BR, 2), lambda i: (i, 0)),
        ],
        out_shape=[
            jax.ShapeDtypeStruct((NC, NACC, DH), _f32),
            jax.ShapeDtypeStruct((NACC, 2), _f32),
        ],
    )(acc[0], acc[1], den.reshape(NACC, 1), b.reshape(1, D), W, A2)


def _tail_body(a0_ref, a1_ref, d_ref, bb_ref, b_ref,
               wl1_ref, bl1_ref, wl2_ref, bl2_ref, out_ref,
               sum_ref, cnt_ref):
    i = pl.program_id(0)
    BR = a0_ref.shape[0]
    den = d_ref[...] + 1e-16
    acc = jnp.concatenate([a0_ref[...], a1_ref[...]], axis=1)
    z = _selu(acc / den + b_ref[...])
    row = i * BR + lax.broadcasted_iota(jnp.int32, z.shape, 0)
    z = jnp.where(row < N, z, 0.0)

    bb = bb_ref[...].reshape(1, BR)
    gid = lax.broadcasted_iota(jnp.int32, (NG, BR), 0)
    msk = (bb == gid).astype(_f32)

    @pl.when(i == 0)
    def _():
        sum_ref[...] = jnp.zeros_like(sum_ref)
        cnt_ref[...] = jnp.zeros_like(cnt_ref)

    sum_ref[...] += jnp.dot(msk, z, preferred_element_type=_f32)
    cnt_ref[...] += jnp.sum(msk, axis=1, keepdims=True)

    @pl.when(i == pl.num_programs(0) - 1)
    def _():
        g = sum_ref[...] / jnp.maximum(cnt_ref[...], 1.0)
        g = _selu(g)
        g = _selu(jnp.dot(g, wl1_ref[...], preferred_element_type=_f32)
                  + bl1_ref[...])
        lg = jnp.dot(g, wl2_ref[...], preferred_element_type=_f32) \
            + bl2_ref[...]
        m = jnp.max(lg, axis=1, keepdims=True)
        lse = jnp.log(jnp.sum(jnp.exp(lg - m), axis=1, keepdims=True))
        out_ref[...] = lg - m - lse


def _tail(acc, den, bb2d, b, Wl1, bl1, Wl2, bl2):
    BR = 1024
    return pl.pallas_call(
        _tail_body,
        grid=(NACC // BR,),
        in_specs=[
            pl.BlockSpec((BR, DH), lambda i: (i, 0)),
            pl.BlockSpec((BR, DH), lambda i: (i, 0)),
            pl.BlockSpec((BR, 1), lambda i: (i, 0)),
            pl.BlockSpec((8, 128), lambda i: (i, 0)),
            pl.BlockSpec((1, D), lambda i: (0, 0)),
            pl.BlockSpec((D, HID), lambda i: (0, 0)),
            pl.BlockSpec((1, HID), lambda i: (0, 0)),
            pl.BlockSpec((HID, DOUT), lambda i: (0, 0)),
            pl.BlockSpec((1, DOUT), lambda i: (0, 0)),
        ],
        out_specs=pl.BlockSpec((NG, DOUT), lambda i: (0, 0)),
        out_shape=jax.ShapeDtypeStruct((NG, DOUT), _f32),
        scratch_shapes=[
            pltpu.VMEM((NG, D), _f32),
            pltpu.VMEM((NG, 1), _f32),
        ],
    )(acc[0], acc[1], den.reshape(NACC, 1), bb2d, b.reshape(1, D),
      Wl1, bl1.reshape(1, HID), Wl2, bl2.reshape(1, DOUT))


def _edge_body(h_hbm, as_hbm, ad_hbm, src_hbm, dst_hbm, zacc_hbm, zden_hbm,
               acc_out, den_out,
               as_v, ad_v, sidx, didx, hrows, exv, acc_sh, den_sh, sem):
    c = lax.axis_index("c")
    s = lax.axis_index("s")

    pltpu.sync_copy(zacc_hbm.at[pl.ds(s * STRIPE, STRIPE)],
                    acc_sh.at[pl.ds(s * STRIPE, STRIPE)])
    pltpu.sync_copy(zden_hbm.at[pl.ds(s * STRIPE, STRIPE)],
                    den_sh.at[pl.ds(s * STRIPE, STRIPE)])
    pltpu.sync_copy(as_hbm, as_v)
    pltpu.sync_copy(ad_hbm, ad_v)
    plsc.subcore_barrier()

    row0 = s * (EW // 128)

    def block(b, carry):
        r = row0 + b * 4
        pltpu.sync_copy(src_hbm.at[pl.ds(r, 4)], sidx)
        pltpu.sync_copy(dst_hbm.at[pl.ds(r, 4)], didx)
        for j in range(4):
            pltpu.async_copy(h_hbm.at[c].at[sidx.at[j]],
                             hrows.at[pl.ds(j * 128, 128)], sem).wait()
        for j in range(4):
            def grp(k, _):
                s16 = sidx[j, pl.ds(k * 16, 16)]
                d16 = didx[j, pl.ds(k * 16, 16)]
                a_s = plsc.load_gather(as_v, [s16])
                a_d = plsc.load_gather(ad_v, [d16])
                e = a_s + a_d
                e = jnp.where(e >= 0.0, e, 0.2 * e)
                exv[pl.ds(j * 128 + k * 16, 16)] = jnp.exp(e)
                return 0
            lax.fori_loop(0, 8, grp, 0, unroll=True)

        def scale(i, _):
            w = plsc.load_gather(exv, [jnp.full((16,), i, jnp.int32)])
            for t in range(DH // 16):
                hrows[i, pl.ds(t * 16, 16)] = hrows[i, pl.ds(t * 16, 16)] * w
            return 0
        lax.fori_loop(0, EB, scale, 0)

        for j in range(4):
            pltpu.sync_copy(hrows.at[pl.ds(j * 128, 128)],
                            acc_sh.at[didx.at[j]], add=True)

        @pl.when(c == 0)
        def _():
            for j in range(4):
                pltpu.sync_copy(exv.at[pl.ds(j * 128, 128)],
                                den_sh.at[didx.at[j]], add=True)
        return carry

    lax.fori_loop(0, NBLK, block, 0)
    plsc.subcore_barrier()

    pltpu.sync_copy(acc_sh.at[pl.ds(s * STRIPE, STRIPE)],
                    acc_out.at[c].at[pl.ds(s * STRIPE, STRIPE)])

    @pl.when(c == 0)
    def _():
        pltpu.sync_copy(den_sh.at[pl.ds(s * STRIPE, STRIPE)],
                        den_out.at[pl.ds(s * STRIPE, STRIPE)])


def _edge_pass(h3, a_s, a_d, src2d, dst2d, zacc, zden):
    mesh = plsc.VectorSubcoreMesh(core_axis_name="c", subcore_axis_name="s")
    f = pl.kernel(
        _edge_body,
        out_type=[
            jax.ShapeDtypeStruct((NC, NACC, DH), _f32),
            jax.ShapeDtypeStruct((NACC,), _f32),
        ],
        mesh=mesh,
        scratch_types=[
            pltpu.VMEM((NACC,), _f32),
            pltpu.VMEM((NACC,), _f32),
            pltpu.VMEM((4, 128), jnp.int32),
            pltpu.VMEM((4, 128), jnp.int32),
            pltpu.VMEM((EB, DH), _f32),
            pltpu.VMEM((EB,), _f32),
            pltpu.VMEM_SHARED((NACC, DH), _f32),
            pltpu.VMEM_SHARED((NACC,), _f32),
            pltpu.SemaphoreType.DMA,
        ],
        compiler_params=pltpu.CompilerParams(needs_layout_passes=False,
                                             use_tc_tiling_on_sc=False),
    )
    return f(h3, a_s, a_d, src2d, dst2d, zacc, zden)


def kernel(x, edge_index, batch, W1, a_src1, a_dst1, b1,
           W2, a_src2, a_dst2, b2, Wl1, bl1, Wl2, bl2):
    loops = jnp.arange(N, dtype=jnp.int32)
    src = jnp.concatenate([edge_index[0].astype(jnp.int32), loops])
    dst = jnp.concatenate([edge_index[1].astype(jnp.int32), loops])
    src2d = jnp.pad(src, (0, ET - src.shape[0])).reshape(ER, 128)
    dst2d = jnp.pad(dst, (0, ET - dst.shape[0]),
                    constant_values=N).reshape(ER, 128)
    xp = jnp.pad(x, ((0, NACC - N), (0, 0)))
    bb2d = jnp.pad(batch.astype(jnp.int32), (0, NACC - N),
                   constant_values=NG).reshape(NACC // 128, 128)
    A1 = jnp.stack([a_src1, a_dst1], axis=1)
    A2 = jnp.stack([a_src2, a_dst2], axis=1)
    zacc = jnp.zeros((NACC, DH), _f32)
    zden = jnp.zeros((NACC,), _f32)

    h1, asad1 = _mm(xp, W1, A1)
    acc1, den1 = _edge_pass(h1, asad1[:, 0], asad1[:, 1], src2d, dst2d,
                            zacc, zden)
    h2, asad2 = _mid(acc1, den1, b1, W2, A2)
    acc2, den2 = _edge_pass(h2, asad2[:, 0], asad2[:, 1], src2d, dst2d,
                            zacc, zden)
    return _tail(acc2, den2, bb2d, b2, Wl1, bl1, Wl2, bl2)


# speedup vs baseline: 20.5766x; 1.6372x over previous
import jax
import jax.numpy as jnp
from jax import lax
from jax.experimental import pallas as pl
from jax.experimental.pallas import tpu as pltpu
from jax.experimental.pallas import tpu_sc as plsc

N = 10000
E = 320000
D = 128
DH = D // 2
HID = 64
DOUT = 32
NG = 64

NC, NS = 2, 16
EB = 256
EBJ = EB // 128
NBLK = 84
EW = EB * NBLK
ET = EW * NS
ER = ET // 128
NACC = 10240
STRIPE = NACC // NS

_f32 = jnp.float32


def _selu(x):
    return 1.0507009873554805 * jnp.where(
        x > 0, x, 1.6732632423543772 * (jnp.exp(x) - 1.0))


def _mm_body(x_ref, w_ref, a2_ref, h_ref, p_ref):
    h = jnp.dot(x_ref[...], w_ref[...], preferred_element_type=_f32)
    h_ref[0] = h[:, :DH]
    h_ref[1] = h[:, DH:]
    p_ref[...] = jnp.dot(h, a2_ref[...], preferred_element_type=_f32)


def _mm(x, W, A2):
    BR = 1024
    return pl.pallas_call(
        _mm_body,
        grid=(NACC // BR,),
        in_specs=[
            pl.BlockSpec((BR, D), lambda i: (i, 0)),
            pl.BlockSpec((D, D), lambda i: (0, 0)),
            pl.BlockSpec((D, 2), lambda i: (0, 0)),
        ],
        out_specs=[
            pl.BlockSpec((NC, BR, DH), lambda i: (0, i, 0)),
            pl.BlockSpec((BR, 2), lambda i: (i, 0)),
        ],
        out_shape=[
            jax.ShapeDtypeStruct((NC, NACC, DH), _f32),
            jax.ShapeDtypeStruct((NACC, 2), _f32),
        ],
    )(x, W, A2)


def _mid_body(a0_ref, a1_ref, d_ref, b_ref, w_ref, a2_ref, h_ref, p_ref):
    i = pl.program_id(0)
    den = d_ref[...] + 1e-16
    acc = jnp.concatenate([a0_ref[...], a1_ref[...]], axis=1)
    z = _selu(acc / den + b_ref[...])
    row = i * a0_ref.shape[0] + lax.broadcasted_iota(jnp.int32, z.shape, 0)
    z = jnp.where(row < N, z, 0.0)
    h = jnp.dot(z, w_ref[...], preferred_element_type=_f32)
    h_ref[0] = h[:, :DH]
    h_ref[1] = h[:, DH:]
    p_ref[...] = jnp.dot(h, a2_ref[...], preferred_element_type=_f32)


def _mid(acc, den, b, W, A2):
    BR = 1024
    return pl.pallas_call(
        _mid_body,
        grid=(NACC // BR,),
        in_specs=[
            pl.BlockSpec((BR, DH), lambda i: (i, 0)),
            pl.BlockSpec((BR, DH), lambda i: (i, 0)),
            pl.BlockSpec((BR, 1), lambda i: (i, 0)),
            pl.BlockSpec((1, D), lambda i: (0, 0)),
            pl.BlockSpec((D, D), lambda i: (0, 0)),
            pl.BlockSpec((D, 2), lambda i: (0, 0)),
        ],
        out_specs=[
            pl.BlockSpec((NC, BR, DH), lambda i: (0, i, 0)),
            pl.BlockSpec((BR, 2), lambda i: (i, 0)),
        ],
        out_shape=[
            jax.ShapeDtypeStruct((NC, NACC, DH), _f32),
            jax.ShapeDtypeStruct((NACC, 2), _f32),
        ],
    )(acc[0], acc[1], den.reshape(NACC, 1), b.reshape(1, D), W, A2)


def _tail_body(a0_ref, a1_ref, d_ref, bb_ref, b_ref,
               wl1_ref, bl1_ref, wl2_ref, bl2_ref, out_ref,
               sum_ref, cnt_ref):
    i = pl.program_id(0)
    BR = a0_ref.shape[0]
    den = d_ref[...] + 1e-16
    acc = jnp.concatenate([a0_ref[...], a1_ref[...]], axis=1)
    z = _selu(acc / den + b_ref[...])
    row = i * BR + lax.broadcasted_iota(jnp.int32, z.shape, 0)
    z = jnp.where(row < N, z, 0.0)

    bb = bb_ref[...].reshape(1, BR)
    gid = lax.broadcasted_iota(jnp.int32, (NG, BR), 0)
    msk = (bb == gid).astype(_f32)

    @pl.when(i == 0)
    def _():
        sum_ref[...] = jnp.zeros_like(sum_ref)
        cnt_ref[...] = jnp.zeros_like(cnt_ref)

    sum_ref[...] += jnp.dot(msk, z, preferred_element_type=_f32)
    cnt_ref[...] += jnp.sum(msk, axis=1, keepdims=True)

    @pl.when(i == pl.num_programs(0) - 1)
    def _():
        g = sum_ref[...] / jnp.maximum(cnt_ref[...], 1.0)
        g = _selu(g)
        g = _selu(jnp.dot(g, wl1_ref[...], preferred_element_type=_f32)
                  + bl1_ref[...])
        lg = jnp.dot(g, wl2_ref[...], preferred_element_type=_f32) \
            + bl2_ref[...]
        m = jnp.max(lg, axis=1, keepdims=True)
        lse = jnp.log(jnp.sum(jnp.exp(lg - m), axis=1, keepdims=True))
        out_ref[...] = lg - m - lse


def _tail(acc, den, bb2d, b, Wl1, bl1, Wl2, bl2):
    BR = 1024
    return pl.pallas_call(
        _tail_body,
        grid=(NACC // BR,),
        in_specs=[
            pl.BlockSpec((BR, DH), lambda i: (i, 0)),
            pl.BlockSpec((BR, DH), lambda i: (i, 0)),
            pl.BlockSpec((BR, 1), lambda i: (i, 0)),
            pl.BlockSpec((8, 128), lambda i: (i, 0)),
            pl.BlockSpec((1, D), lambda i: (0, 0)),
            pl.BlockSpec((D, HID), lambda i: (0, 0)),
            pl.BlockSpec((1, HID), lambda i: (0, 0)),
            pl.BlockSpec((HID, DOUT), lambda i: (0, 0)),
            pl.BlockSpec((1, DOUT), lambda i: (0, 0)),
        ],
        out_specs=pl.BlockSpec((NG, DOUT), lambda i: (0, 0)),
        out_shape=jax.ShapeDtypeStruct((NG, DOUT), _f32),
        scratch_shapes=[
            pltpu.VMEM((NG, D), _f32),
            pltpu.VMEM((NG, 1), _f32),
        ],
    )(acc[0], acc[1], den.reshape(NACC, 1), bb2d, b.reshape(1, D),
      Wl1, bl1.reshape(1, HID), Wl2, bl2.reshape(1, DOUT))


def _edge_body(h_hbm, as_hbm, ad_hbm, src_hbm, dst_hbm, zacc_hbm, zden_hbm,
               acc_out, den_out,
               as_v, ad_v, sidx, didx, hrows, exv, acc_sh, den_sh,
               gsem, ssem, isem):
    c = lax.axis_index("c")
    s = lax.axis_index("s")

    pltpu.sync_copy(zacc_hbm.at[pl.ds(s * STRIPE, STRIPE)],
                    acc_sh.at[pl.ds(s * STRIPE, STRIPE)])
    pltpu.sync_copy(zden_hbm.at[pl.ds(s * STRIPE, STRIPE)],
                    den_sh.at[pl.ds(s * STRIPE, STRIPE)])
    pltpu.sync_copy(as_hbm, as_v)
    pltpu.sync_copy(ad_hbm, ad_v)
    plsc.subcore_barrier()

    row0 = s * (EW // 128)

    def fire_idx(b, slot, sync):
        r = row0 + b * EBJ
        if sync:
            pltpu.sync_copy(src_hbm.at[pl.ds(r, EBJ)], sidx.at[slot])
            pltpu.sync_copy(dst_hbm.at[pl.ds(r, EBJ)], didx.at[slot])
        else:
            pltpu.async_copy(src_hbm.at[pl.ds(r, EBJ)], sidx.at[slot], isem)
            pltpu.async_copy(dst_hbm.at[pl.ds(r, EBJ)], didx.at[slot], isem)

    def drain_idx():
        for _ in range(2):
            pltpu.make_async_copy(src_hbm.at[pl.ds(0, EBJ)],
                                  sidx.at[0], isem).wait()

    def fire_gather(slot4, slot3):
        for j in range(EBJ):
            pltpu.async_copy(h_hbm.at[c].at[sidx.at[slot4].at[j]],
                             hrows.at[slot3].at[pl.ds(j * 128, 128)], gsem)

    def drain_gather():
        pltpu.make_async_copy(zacc_hbm.at[pl.ds(0, EB)],
                              hrows.at[0], gsem).wait()

    def fire_scatter(slot4, slot3):
        for j in range(EBJ):
            pltpu.async_copy(hrows.at[slot3].at[pl.ds(j * 128, 128)],
                             acc_sh.at[didx.at[slot4].at[j]], ssem, add=True)

        @pl.when(c == 0)
        def _():
            for j in range(EBJ):
                pltpu.async_copy(exv.at[pl.ds(slot3 * EB + j * 128, 128)],
                                 den_sh.at[didx.at[slot4].at[j]], ssem,
                                 add=True)

    def drain_scatter():
        pltpu.make_async_copy(zacc_hbm.at[pl.ds(0, EB)],
                              acc_sh.at[pl.ds(0, EB)], ssem).wait()

        @pl.when(c == 0)
        def _():
            pltpu.make_async_copy(zden_hbm.at[pl.ds(0, EB)],
                                  den_sh.at[pl.ds(0, EB)], ssem).wait()

    fire_idx(0, 0, True)
    fire_idx(1, 1, False)
    fire_gather(0, 0)

    def block(b, carry):
        p3 = b % 3
        p4 = b % 4

        @pl.when(b >= 2)
        def _():
            drain_scatter()

        @pl.when(b + 2 < NBLK)
        def _():
            fire_idx(b + 2, (b + 2) % 4, False)

        @pl.when(b + 1 < NBLK)
        def _():
            drain_idx()
            fire_gather((b + 1) % 4, (b + 1) % 3)

        drain_gather()

        for j in range(EBJ):
            def grp(k, _):
                s16 = sidx[p4, j, pl.ds(k * 16, 16)]
                d16 = didx[p4, j, pl.ds(k * 16, 16)]
                a_s = plsc.load_gather(as_v, [s16])
                a_d = plsc.load_gather(ad_v, [d16])
                e = a_s + a_d
                e = jnp.where(e >= 0.0, e, 0.2 * e)
                exv[pl.ds(p3 * EB + j * 128 + k * 16, 16)] = jnp.exp(e)
                return 0
            lax.fori_loop(0, 8, grp, 0, unroll=True)

        def scale(i, _):
            w = plsc.load_gather(exv, [jnp.full((16,), p3 * EB + i,
                                                jnp.int32)])
            for t in range(DH // 16):
                hrows[p3, i, pl.ds(t * 16, 16)] = \
                    hrows[p3, i, pl.ds(t * 16, 16)] * w
            return 0
        lax.fori_loop(0, EB, scale, 0, unroll=4)

        fire_scatter(p4, p3)
        return carry

    lax.fori_loop(0, NBLK, block, 0)
    drain_scatter()
    drain_scatter()
    plsc.subcore_barrier()

    pltpu.sync_copy(acc_sh.at[pl.ds(s * STRIPE, STRIPE)],
                    acc_out.at[c].at[pl.ds(s * STRIPE, STRIPE)])

    @pl.when(c == 0)
    def _():
        pltpu.sync_copy(den_sh.at[pl.ds(s * STRIPE, STRIPE)],
                        den_out.at[pl.ds(s * STRIPE, STRIPE)])


def _edge_pass(h3, a_s, a_d, src2d, dst2d, zacc, zden):
    mesh = plsc.VectorSubcoreMesh(core_axis_name="c", subcore_axis_name="s")
    f = pl.kernel(
        _edge_body,
        out_type=[
            jax.ShapeDtypeStruct((NC, NACC, DH), _f32),
            jax.ShapeDtypeStruct((NACC,), _f32),
        ],
        mesh=mesh,
        scratch_types=[
            pltpu.VMEM((NACC,), _f32),
            pltpu.VMEM((NACC,), _f32),
            pltpu.VMEM((4, EBJ, 128), jnp.int32),
            pltpu.VMEM((4, EBJ, 128), jnp.int32),
            pltpu.VMEM((3, EB, DH), _f32),
            pltpu.VMEM((3 * EB,), _f32),
            pltpu.VMEM_SHARED((NACC, DH), _f32),
            pltpu.VMEM_SHARED((NACC,), _f32),
            pltpu.SemaphoreType.DMA,
            pltpu.SemaphoreType.DMA,
            pltpu.SemaphoreType.DMA,
        ],
        compiler_params=pltpu.CompilerParams(needs_layout_passes=False,
                                             use_tc_tiling_on_sc=False),
    )
    return f(h3, a_s, a_d, src2d, dst2d, zacc, zden)


def kernel(x, edge_index, batch, W1, a_src1, a_dst1, b1,
           W2, a_src2, a_dst2, b2, Wl1, bl1, Wl2, bl2):
    loops = jnp.arange(N, dtype=jnp.int32)
    src = jnp.concatenate([edge_index[0].astype(jnp.int32), loops])
    dst = jnp.concatenate([edge_index[1].astype(jnp.int32), loops])
    src2d = jnp.pad(src, (0, ET - src.shape[0])).reshape(ER, 128)
    dst2d = jnp.pad(dst, (0, ET - dst.shape[0]),
                    constant_values=N).reshape(ER, 128)
    xp = jnp.pad(x, ((0, NACC - N), (0, 0)))
    bb2d = jnp.pad(batch.astype(jnp.int32), (0, NACC - N),
                   constant_values=NG).reshape(NACC // 128, 128)
    A1 = jnp.stack([a_src1, a_dst1], axis=1)
    A2 = jnp.stack([a_src2, a_dst2], axis=1)
    zacc = jnp.zeros((NACC, DH), _f32)
    zden = jnp.zeros((NACC,), _f32)

    h1, asad1 = _mm(xp, W1, A1)
    acc1, den1 = _edge_pass(h1, asad1[:, 0], asad1[:, 1], src2d, dst2d,
                            zacc, zden)
    h2, asad2 = _mid(acc1, den1, b1, W2, A2)
    acc2, den2 = _edge_pass(h2, asad2[:, 0], asad2[:, 1], src2d, dst2d,
                            zacc, zden)
    return _tail(acc2, den2, bb2d, b2, Wl1, bl1, Wl2, bl2)
